# baseline (speedup 1.0000x reference)
import jax
import jax.numpy as jnp
from jax.experimental import pallas as pl

N = 10000
E = 320000
F_IN = 128
HID = 64
HEADS = 8
NCOMP = 32
OUT = 10


def _gat(x, src, dst, W, a_src, a_dst, b):
    n = x.shape[0]
    xp = (x @ W).reshape(n, HEADS, HID)
    al_s = (xp * a_src[None, :, :]).sum(-1)
    al_d = (xp * a_dst[None, :, :]).sum(-1)
    e = al_s[src] + al_d[dst]
    e = jnp.where(e > 0, e, 0.2 * e)
    m = jax.ops.segment_max(e, dst, num_segments=n)
    ex = jnp.exp(e - m[dst])
    s = jax.ops.segment_sum(ex, dst, num_segments=n)
    alpha = ex / (s[dst] + 1e-16)
    out = jax.ops.segment_sum(alpha[:, :, None] * xp[src], dst, num_segments=n)
    return out.reshape(n, HEADS * HID) + b


def _gcn(x, src, dst, W, b):
    n = x.shape[0]
    deg = jax.ops.segment_sum(jnp.ones_like(src, dtype=x.dtype), dst, num_segments=n)
    dinv = jnp.where(deg > 0, deg ** -0.5, 0.0)
    norm = dinv[src] * dinv[dst]
    h = x @ W
    out = jax.ops.segment_sum(norm[:, None] * h[src], dst, num_segments=n)
    return out + b


def _gnn(x, edge_index, p):
    n = x.shape[0]
    loop = jnp.arange(n, dtype=edge_index.dtype)
    src = jnp.concatenate([edge_index[0], loop])
    dst = jnp.concatenate([edge_index[1], loop])
    h = jax.nn.elu(_gat(x, src, dst, p["gat_w"], p["att_src"], p["att_dst"], p["gat_b"]))
    h = jax.nn.elu(h @ p["lin_w"] + p["lin_b"])
    h = jax.nn.elu(_gcn(h, src, dst, p["gcn1_w"], p["gcn1_b"]))
    h = _gcn(h, src, dst, p["gcn2_w"], p["gcn2_b"])
    return h


def _sym_decorr(W):
    s, u = jnp.linalg.eigh(W @ W.T)
    return (u * (1.0 / jnp.sqrt(jnp.maximum(s, 1e-12)))[None, :]) @ u.T @ W


def _fastica(X, n_components):
    Nn = X.shape[0]
    Xc = X - X.mean(axis=0, keepdims=True)
    cov = (Xc.T @ Xc) / Nn
    ev, EV = jnp.linalg.eigh(cov)
    idx = jnp.argsort(ev)[::-1][:n_components]
    d = jnp.maximum(ev[idx], 1e-12)
    K = (EV[:, idx] / jnp.sqrt(d)[None, :]).T
    Xw = Xc @ K.T
    W = jax.random.normal(jax.random.key(42), (n_components, n_components), dtype=X.dtype)
    W = _sym_decorr(W)
    for _ in range(50):
        WX = Xw @ W.T
        g = jnp.tanh(WX)
        gp = (1.0 - g ** 2).mean(axis=0)
        W = _sym_decorr((g.T @ Xw) / Nn - gp[:, None] * W)
    return Xw @ W.T


def _clf_body(src_ref, w_ref, b_ref, out_ref):
    logits = jnp.dot(src_ref[...], w_ref[...], preferred_element_type=jnp.float32)
    logits = logits + b_ref[...][None, :]
    mx = jnp.max(logits, axis=1, keepdims=True)
    sh = logits - mx
    lse = jnp.log(jnp.sum(jnp.exp(sh), axis=1, keepdims=True))
    out_ref[...] = sh - lse


def _clf_head(src, w, b):
    return pl.pallas_call(
        _clf_body,
        out_shape=jax.ShapeDtypeStruct((src.shape[0], w.shape[1]), jnp.float32),
    )(src, w, b)


def kernel(ct_x, ct_edge_index, pet_x, pet_edge_index, fused_x, fused_edge_index, params):
    ct = _gnn(ct_x, ct_edge_index, params["ct"])
    pet = _gnn(pet_x, pet_edge_index, params["pet"])
    fu = _gnn(fused_x, fused_edge_index, params["fused"])
    cat = jax.lax.stop_gradient(jnp.concatenate([ct, pet, fu], axis=1))
    src = _fastica(cat, NCOMP)
    return _clf_head(src, params["clf_w"], params["clf_b"])


# speedup vs baseline: 2.3186x; 2.3186x over previous
import jax
import jax.numpy as jnp
from jax.experimental import pallas as pl

N = 10000
E = 320000
F_IN = 128
HID = 64
HEADS = 8
NCOMP = 32
OUT = 10


_FOLD_BLK = 2048


def _fold_body(dst_ref, upd_ref, out_ref, acc_ref, prev_ref):
    @pl.when(pl.program_id(0) == 0)
    def _():
        prev_ref[0] = -1

    def body(i, _):
        dcur = dst_ref[0, 0, i]
        prev = prev_ref[0]
        row = upd_ref[0, pl.ds(i, 1), :]
        changed = dcur != prev

        @pl.when(changed & (prev >= 0) & (prev < N))
        def _():
            out_ref[pl.ds(prev, 1), :] = acc_ref[...]

        acc_ref[...] = jnp.where(changed, row, acc_ref[...] + row)
        prev_ref[0] = dcur
        return 0

    jax.lax.fori_loop(0, _FOLD_BLK, body, 0)


def _fold_segment_sum(upd, dst, n):
    en, d = upd.shape
    perm = jnp.argsort(dst, stable=True)
    upd_s = upd[perm]
    dst_s = dst[perm].astype(jnp.int32)
    nblk = (en + _FOLD_BLK - 1) // _FOLD_BLK
    pad = nblk * _FOLD_BLK - en
    upd_s = jnp.pad(upd_s, ((0, pad), (0, 0)))
    dst_s = jnp.pad(dst_s, (0, pad), constant_values=jnp.int32(2 ** 30))
    from jax.experimental.pallas import tpu as pltpu
    return pl.pallas_call(
        _fold_body,
        grid=(nblk,),
        in_specs=[
            pl.BlockSpec((1, 1, _FOLD_BLK), lambda i: (i, 0, 0),
                         memory_space=pltpu.SMEM),
            pl.BlockSpec((1, _FOLD_BLK, d), lambda i: (i, 0, 0)),
        ],
        out_specs=pl.BlockSpec((n, d), lambda i: (0, 0)),
        out_shape=jax.ShapeDtypeStruct((n, d), jnp.float32),
        scratch_shapes=[
            pltpu.VMEM((1, d), jnp.float32),
            pltpu.SMEM((1,), jnp.int32),
        ],
    )(dst_s.reshape(nblk, 1, _FOLD_BLK), upd_s.reshape(nblk, _FOLD_BLK, d))


def _gat(x, src, dst, W, a_src, a_dst, b):
    n = x.shape[0]
    xp = (x @ W).reshape(n, HEADS, HID)
    al_s = (xp * a_src[None, :, :]).sum(-1)
    al_d = (xp * a_dst[None, :, :]).sum(-1)
    e = al_s[src] + al_d[dst]
    e = jnp.where(e > 0, e, 0.2 * e)
    m = jax.ops.segment_max(e, dst, num_segments=n)
    ex = jnp.exp(e - m[dst])
    s = jax.ops.segment_sum(ex, dst, num_segments=n)
    alpha = ex / (s[dst] + 1e-16)
    upd = (alpha[:, :, None] * xp[src]).reshape(-1, HEADS * HID)
    out = _fold_segment_sum(upd, dst, n)
    return out + b


def _gcn(x, src, dst, W, b):
    n = x.shape[0]
    deg = jax.ops.segment_sum(jnp.ones_like(src, dtype=x.dtype), dst, num_segments=n)
    dinv = jnp.where(deg > 0, deg ** -0.5, 0.0)
    norm = dinv[src] * dinv[dst]
    h = x @ W
    out = jax.ops.segment_sum(norm[:, None] * h[src], dst, num_segments=n)
    return out + b


def _gnn(x, edge_index, p):
    n = x.shape[0]
    loop = jnp.arange(n, dtype=edge_index.dtype)
    src = jnp.concatenate([edge_index[0], loop])
    dst = jnp.concatenate([edge_index[1], loop])
    h = jax.nn.elu(_gat(x, src, dst, p["gat_w"], p["att_src"], p["att_dst"], p["gat_b"]))
    h = jax.nn.elu(h @ p["lin_w"] + p["lin_b"])
    h = jax.nn.elu(_gcn(h, src, dst, p["gcn1_w"], p["gcn1_b"]))
    h = _gcn(h, src, dst, p["gcn2_w"], p["gcn2_b"])
    return h


def _sym_decorr(W):
    s, u = jnp.linalg.eigh(W @ W.T)
    return (u * (1.0 / jnp.sqrt(jnp.maximum(s, 1e-12)))[None, :]) @ u.T @ W


def _fastica(X, n_components):
    Nn = X.shape[0]
    Xc = X - X.mean(axis=0, keepdims=True)
    cov = (Xc.T @ Xc) / Nn
    ev, EV = jnp.linalg.eigh(cov)
    idx = jnp.argsort(ev)[::-1][:n_components]
    d = jnp.maximum(ev[idx], 1e-12)
    K = (EV[:, idx] / jnp.sqrt(d)[None, :]).T
    Xw = Xc @ K.T
    W = jax.random.normal(jax.random.key(42), (n_components, n_components), dtype=X.dtype)
    W = _sym_decorr(W)
    for _ in range(50):
        WX = Xw @ W.T
        g = jnp.tanh(WX)
        gp = (1.0 - g ** 2).mean(axis=0)
        W = _sym_decorr((g.T @ Xw) / Nn - gp[:, None] * W)
    return Xw @ W.T


def _clf_body(src_ref, w_ref, b_ref, out_ref):
    logits = jnp.dot(src_ref[...], w_ref[...], preferred_element_type=jnp.float32)
    logits = logits + b_ref[...][None, :]
    mx = jnp.max(logits, axis=1, keepdims=True)
    sh = logits - mx
    lse = jnp.log(jnp.sum(jnp.exp(sh), axis=1, keepdims=True))
    out_ref[...] = sh - lse


def _clf_head(src, w, b):
    return pl.pallas_call(
        _clf_body,
        out_shape=jax.ShapeDtypeStruct((src.shape[0], w.shape[1]), jnp.float32),
    )(src, w, b)


def kernel(ct_x, ct_edge_index, pet_x, pet_edge_index, fused_x, fused_edge_index, params):
    ct = _gnn(ct_x, ct_edge_index, params["ct"])
    pet = _gnn(pet_x, pet_edge_index, params["pet"])
    fu = _gnn(fused_x, fused_edge_index, params["fused"])
    cat = jax.lax.stop_gradient(jnp.concatenate([ct, pet, fu], axis=1))
    src = _fastica(cat, NCOMP)
    return _clf_head(src, params["clf_w"], params["clf_b"])
